# baseline (speedup 1.0000x reference)
import functools

import jax
import jax.numpy as jnp
from jax import lax
from jax.experimental import pallas as pl
from jax.experimental.pallas import tpu as pltpu
from jax.experimental.pallas import tpu_sc as plsc

N_NODES = 10000
D_FEAT = 128
N_EDGES = 320000

NC = 2
NS = 16
NW = NC * NS
EPW = N_EDGES // NW
GS = 125
NR = EPW // GS
NGS = 8
NCH = NR // NGS
NVREG = D_FEAT // 16
NFULL = GS // 16
NTAIL = GS - NFULL * 16

ROWS_A = 632
ROWS_LAST = N_NODES - (NS - 1) * ROWS_A

_mesh = plsc.VectorSubcoreMesh(core_axis_name="c", subcore_axis_name="s")


@functools.partial(
    pl.kernel,
    out_type=jax.ShapeDtypeStruct((NC, N_NODES, D_FEAT), jnp.float32),
    mesh=_mesh,
    scratch_types=[
        pltpu.VMEM((NGS, GS), jnp.int32),
        pltpu.VMEM((NGS, GS), jnp.int32),
        pltpu.VMEM((NGS, GS), jnp.float32),
        pltpu.VMEM((2, 128, D_FEAT), jnp.float32),
        pltpu.VMEM_SHARED((N_NODES, D_FEAT), jnp.float32),
        pltpu.SemaphoreType.DMA,
        pltpu.SemaphoreType.DMA,
    ],
)
def _sc_aggregate(x_hbm, src_hbm, dst_hbm, w_hbm, out_hbm,
                  src_v, dst_v, w_v, rows_v, acc_sh, gsem, ssem):
    c = lax.axis_index("c")
    s = lax.axis_index("s")
    wid = c * NS + s
    rbase = s * ROWS_A

    zero16 = jnp.zeros((16,), jnp.float32)

    def _zero_row(i, carry):
        for j in range(NVREG):
            rows_v[0, i, pl.ds(j * 16, 16)] = zero16
        return carry

    lax.fori_loop(0, 128, _zero_row, 0)
    for t in range(4):
        pltpu.sync_copy(rows_v.at[0, pl.ds(0, 128)],
                        acc_sh.at[pl.ds(rbase + t * 128, 128)])

    @pl.when(s < NS - 1)
    def _zero_rest():
        pltpu.sync_copy(rows_v.at[0, pl.ds(0, ROWS_A - 512)],
                        acc_sh.at[pl.ds(rbase + 512, ROWS_A - 512)])

    @pl.when(s == NS - 1)
    def _zero_rest_last():
        pltpu.sync_copy(rows_v.at[0, pl.ds(0, ROWS_LAST - 512)],
                        acc_sh.at[pl.ds(rbase + 512, ROWS_LAST - 512)])

    plsc.subcore_barrier()

    bcast_idx = [jnp.full((16, 1), k, jnp.int32) for k in range(16)]
    gdims = lax.GatherDimensionNumbers(
        offset_dims=(), collapsed_slice_dims=(0,), start_index_map=(0,))

    def _bcast(wv, k):
        return lax.gather(wv, bcast_idx[k], dimension_numbers=gdims,
                          slice_sizes=(1,),
                          mode=lax.GatherScatterMode.PROMISE_IN_BOUNDS)

    def _scale16(p, ebase, wv, ks):
        for k in ks:
            wk = _bcast(wv, k)
            e = ebase + k
            for j in range(NVREG):
                rows_v[p, e, pl.ds(j * 16, 16)] = (
                    rows_v[p, e, pl.ds(j * 16, 16)] * wk)

    def _chunk(ci, carry):
        pltpu.sync_copy(src_hbm.at[wid, pl.ds(ci * NGS, NGS)], src_v)
        pltpu.sync_copy(dst_hbm.at[wid, pl.ds(ci * NGS, NGS)], dst_v)
        pltpu.sync_copy(w_hbm.at[wid, pl.ds(ci * NGS, NGS)], w_v)

        pltpu.async_copy(x_hbm.at[src_v.at[0]],
                         rows_v.at[0, pl.ds(0, GS)], gsem)

        def _scale_stream(p, k):
            def _scale_group(g, acc2):
                wv = w_v[k, pl.ds(g * 16, 16)]
                _scale16(p, g * 16, wv, range(16))
                return acc2

            lax.fori_loop(0, NFULL, _scale_group, 0)
            wv_t = w_v[k, pl.ds(GS - 16, 16)]
            _scale16(p, GS - 16, wv_t, range(16 - NTAIL, 16))

        def _pair(h, inner):
            k0 = 2 * h
            k1 = 2 * h + 1
            pltpu.make_async_copy(x_hbm.at[src_v.at[k0]],
                                  rows_v.at[0, pl.ds(0, GS)], gsem).wait()

            @pl.when(h == 0)
            def _prime_b1():
                pltpu.async_copy(x_hbm.at[src_v.at[k1]],
                                 rows_v.at[1, pl.ds(0, GS)], gsem)

            _scale_stream(0, k0)
            pltpu.async_copy(rows_v.at[0, pl.ds(0, GS)],
                             acc_sh.at[dst_v.at[k0]], ssem, add=True)
            pltpu.make_async_copy(x_hbm.at[src_v.at[k1]],
                                  rows_v.at[1, pl.ds(0, GS)], gsem).wait()
            _scale_stream(1, k1)
            pltpu.async_copy(rows_v.at[1, pl.ds(0, GS)],
                             acc_sh.at[dst_v.at[k1]], ssem, add=True)
            pltpu.make_async_copy(rows_v.at[0, pl.ds(0, GS)],
                                  acc_sh.at[dst_v.at[k0]], ssem).wait()

            @pl.when(h < NGS // 2 - 1)
            def _prefetch0():
                pltpu.async_copy(x_hbm.at[src_v.at[k0 + 2]],
                                 rows_v.at[0, pl.ds(0, GS)], gsem)

            pltpu.make_async_copy(rows_v.at[1, pl.ds(0, GS)],
                                  acc_sh.at[dst_v.at[k1]], ssem).wait()

            @pl.when(h < NGS // 2 - 1)
            def _prefetch1():
                pltpu.async_copy(x_hbm.at[src_v.at[k1 + 2]],
                                 rows_v.at[1, pl.ds(0, GS)], gsem)
            return inner

        lax.fori_loop(0, NGS // 2, _pair, 0)
        return carry

    lax.fori_loop(0, NCH, _chunk, 0)
    plsc.subcore_barrier()

    for t in range(4):
        pltpu.sync_copy(acc_sh.at[pl.ds(rbase + t * 128, 128)],
                        out_hbm.at[c, pl.ds(rbase + t * 128, 128)])

    @pl.when(s < NS - 1)
    def _drain_rest():
        pltpu.sync_copy(acc_sh.at[pl.ds(rbase + 512, ROWS_A - 512)],
                        out_hbm.at[c, pl.ds(rbase + 512, ROWS_A - 512)])

    @pl.when(s == NS - 1)
    def _drain_rest_last():
        pltpu.sync_copy(acc_sh.at[pl.ds(rbase + 512, ROWS_LAST - 512)],
                        out_hbm.at[c, pl.ds(rbase + 512, ROWS_LAST - 512)])


_TC_BLOCK = 1000


def _tc_body(p0_ref, p1_ref, w_ref, b_ref, o_ref):
    acc = p0_ref[...] + p1_ref[...]
    y = jnp.dot(acc, w_ref[...], preferred_element_type=jnp.float32)
    o_ref[...] = jnp.maximum(y + b_ref[...], 0.0)


_tc_finish = pl.pallas_call(
    _tc_body,
    grid=(N_NODES // _TC_BLOCK,),
    in_specs=[
        pl.BlockSpec((_TC_BLOCK, D_FEAT), lambda i: (i, 0)),
        pl.BlockSpec((_TC_BLOCK, D_FEAT), lambda i: (i, 0)),
        pl.BlockSpec((D_FEAT, D_FEAT), lambda i: (0, 0)),
        pl.BlockSpec((1, D_FEAT), lambda i: (0, 0)),
    ],
    out_specs=pl.BlockSpec((_TC_BLOCK, D_FEAT), lambda i: (i, 0)),
    out_shape=jax.ShapeDtypeStruct((N_NODES, D_FEAT), jnp.float32),
)


def kernel(node_features, edge_index, edge_weight, W, b):
    ei = edge_index.astype(jnp.int32)
    dst3d = ei[0].reshape(NW, NR, GS)
    src3d = ei[1].reshape(NW, NR, GS)
    w3d = edge_weight.reshape(NW, NR, GS)
    partials = _sc_aggregate(node_features, src3d, dst3d, w3d)
    return _tc_finish(partials[0], partials[1], W, b.reshape(1, D_FEAT))


# speedup vs baseline: 1.1171x; 1.1171x over previous
import functools

import jax
import jax.numpy as jnp
from jax import lax
from jax.experimental import pallas as pl
from jax.experimental.pallas import tpu as pltpu
from jax.experimental.pallas import tpu_sc as plsc

N_NODES = 10000
D_FEAT = 128
N_EDGES = 320000

NC = 2
NS = 16
NW = NC * NS
EPW = N_EDGES // NW
GS = 125
NR = EPW // GS
NGS = 8
NCH = NR // NGS
NVREG = D_FEAT // 16
NFULL = GS // 16
NTAIL = GS - NFULL * 16

ROWS_A = 632
ROWS_LAST = N_NODES - (NS - 1) * ROWS_A

_mesh = plsc.VectorSubcoreMesh(core_axis_name="c", subcore_axis_name="s")


@functools.partial(
    pl.kernel,
    out_type=jax.ShapeDtypeStruct((NC, N_NODES, D_FEAT), jnp.float32),
    mesh=_mesh,
    scratch_types=[
        pltpu.VMEM((NR, GS), jnp.int32),
        pltpu.VMEM((NGS, GS), jnp.int32),
        pltpu.VMEM((NGS, GS), jnp.float32),
        pltpu.VMEM((2, 128, D_FEAT), jnp.float32),
        pltpu.VMEM_SHARED((N_NODES, D_FEAT), jnp.float32),
        pltpu.SemaphoreType.DMA,
        pltpu.SemaphoreType.DMA,
    ],
)
def _sc_aggregate(x_hbm, src_hbm, dst_hbm, w_hbm, out_hbm,
                  src_v, dst_v, w_v, rows_v, acc_sh, gsem, ssem):
    c = lax.axis_index("c")
    s = lax.axis_index("s")
    wid = c * NS + s
    rbase = s * ROWS_A

    zero16 = jnp.zeros((16,), jnp.float32)

    def _zero_row(i, carry):
        for j in range(NVREG):
            rows_v[0, i, pl.ds(j * 16, 16)] = zero16
        return carry

    lax.fori_loop(0, 128, _zero_row, 0)
    for t in range(4):
        pltpu.sync_copy(rows_v.at[0, pl.ds(0, 128)],
                        acc_sh.at[pl.ds(rbase + t * 128, 128)])

    @pl.when(s < NS - 1)
    def _zero_rest():
        pltpu.sync_copy(rows_v.at[0, pl.ds(0, ROWS_A - 512)],
                        acc_sh.at[pl.ds(rbase + 512, ROWS_A - 512)])

    @pl.when(s == NS - 1)
    def _zero_rest_last():
        pltpu.sync_copy(rows_v.at[0, pl.ds(0, ROWS_LAST - 512)],
                        acc_sh.at[pl.ds(rbase + 512, ROWS_LAST - 512)])

    plsc.subcore_barrier()

    bcast_idx = [jnp.full((16, 1), k, jnp.int32) for k in range(16)]
    gdims = lax.GatherDimensionNumbers(
        offset_dims=(), collapsed_slice_dims=(0,), start_index_map=(0,))

    def _bcast(wv, k):
        return lax.gather(wv, bcast_idx[k], dimension_numbers=gdims,
                          slice_sizes=(1,),
                          mode=lax.GatherScatterMode.PROMISE_IN_BOUNDS)

    def _scale16(p, ebase, wv, ks):
        for k in ks:
            wk = _bcast(wv, k)
            e = ebase + k
            for j in range(NVREG):
                rows_v[p, e, pl.ds(j * 16, 16)] = (
                    rows_v[p, e, pl.ds(j * 16, 16)] * wk)

    def _scale_stream(p, l):
        def _scale_group(g, acc2):
            wv = w_v[l, pl.ds(g * 16, 16)]
            _scale16(p, g * 16, wv, range(16))
            return acc2

        lax.fori_loop(0, NFULL, _scale_group, 0)
        wv_t = w_v[l, pl.ds(GS - 16, 16)]
        _scale16(p, GS - 16, wv_t, range(16 - NTAIL, 16))

    pltpu.sync_copy(src_hbm.at[wid], src_v)
    pltpu.async_copy(x_hbm.at[src_v.at[0]], rows_v.at[0, pl.ds(0, GS)],
                     gsem)
    pltpu.async_copy(x_hbm.at[src_v.at[1]], rows_v.at[1, pl.ds(0, GS)],
                     gsem)

    NPAIR = NR // 2

    def _pair(h, inner):
        ci = h // (NGS // 2)
        l0 = 2 * (h % (NGS // 2))
        l1 = l0 + 1
        k0 = 2 * h
        k1 = k0 + 1

        @pl.when(l0 == 0)
        def _load_idx_chunk():
            pltpu.sync_copy(dst_hbm.at[wid, pl.ds(ci * NGS, NGS)], dst_v)
            pltpu.sync_copy(w_hbm.at[wid, pl.ds(ci * NGS, NGS)], w_v)

        pltpu.make_async_copy(x_hbm.at[src_v.at[k0]],
                              rows_v.at[0, pl.ds(0, GS)], gsem).wait()
        _scale_stream(0, l0)
        pltpu.sync_copy(rows_v.at[0, pl.ds(0, GS)],
                        acc_sh.at[dst_v.at[l0]], add=True)

        @pl.when(h < NPAIR - 1)
        def _prefetch0():
            pltpu.async_copy(x_hbm.at[src_v.at[k0 + 2]],
                             rows_v.at[0, pl.ds(0, GS)], gsem)

        pltpu.make_async_copy(x_hbm.at[src_v.at[k1]],
                              rows_v.at[1, pl.ds(0, GS)], gsem).wait()
        _scale_stream(1, l1)
        pltpu.sync_copy(rows_v.at[1, pl.ds(0, GS)],
                        acc_sh.at[dst_v.at[l1]], add=True)

        @pl.when(h < NPAIR - 1)
        def _prefetch1():
            pltpu.async_copy(x_hbm.at[src_v.at[k1 + 2]],
                             rows_v.at[1, pl.ds(0, GS)], gsem)
        return inner

    lax.fori_loop(0, NPAIR, _pair, 0)
    plsc.subcore_barrier()

    for t in range(4):
        pltpu.sync_copy(acc_sh.at[pl.ds(rbase + t * 128, 128)],
                        out_hbm.at[c, pl.ds(rbase + t * 128, 128)])

    @pl.when(s < NS - 1)
    def _drain_rest():
        pltpu.sync_copy(acc_sh.at[pl.ds(rbase + 512, ROWS_A - 512)],
                        out_hbm.at[c, pl.ds(rbase + 512, ROWS_A - 512)])

    @pl.when(s == NS - 1)
    def _drain_rest_last():
        pltpu.sync_copy(acc_sh.at[pl.ds(rbase + 512, ROWS_LAST - 512)],
                        out_hbm.at[c, pl.ds(rbase + 512, ROWS_LAST - 512)])


_TC_BLOCK = 1000


def _tc_body(p0_ref, p1_ref, w_ref, b_ref, o_ref):
    acc = p0_ref[...] + p1_ref[...]
    y = jnp.dot(acc, w_ref[...], preferred_element_type=jnp.float32)
    o_ref[...] = jnp.maximum(y + b_ref[...], 0.0)


_tc_finish = pl.pallas_call(
    _tc_body,
    grid=(N_NODES // _TC_BLOCK,),
    in_specs=[
        pl.BlockSpec((_TC_BLOCK, D_FEAT), lambda i: (i, 0)),
        pl.BlockSpec((_TC_BLOCK, D_FEAT), lambda i: (i, 0)),
        pl.BlockSpec((D_FEAT, D_FEAT), lambda i: (0, 0)),
        pl.BlockSpec((1, D_FEAT), lambda i: (0, 0)),
    ],
    out_specs=pl.BlockSpec((_TC_BLOCK, D_FEAT), lambda i: (i, 0)),
    out_shape=jax.ShapeDtypeStruct((N_NODES, D_FEAT), jnp.float32),
)


def kernel(node_features, edge_index, edge_weight, W, b):
    ei = edge_index.astype(jnp.int32)
    dst3d = ei[0].reshape(NW, NR, GS)
    src3d = ei[1].reshape(NW, NR, GS)
    w3d = edge_weight.reshape(NW, NR, GS)
    partials = _sc_aggregate(node_features, src3d, dst3d, w3d)
    return _tc_finish(partials[0], partials[1], W, b.reshape(1, D_FEAT))


# speedup vs baseline: 1.1350x; 1.0160x over previous
import functools

import jax
import jax.numpy as jnp
from jax import lax
from jax.experimental import pallas as pl
from jax.experimental.pallas import tpu as pltpu
from jax.experimental.pallas import tpu_sc as plsc

N_NODES = 10000
D_FEAT = 128
N_EDGES = 320000

NC = 2
NS = 16
NW = NC * NS
EPW = N_EDGES // NW
GS = 125
NR = EPW // GS
NGS = 8
NCH = NR // NGS
NVREG = D_FEAT // 16
NFULL = GS // 16
NTAIL = GS - NFULL * 16

ROWS_A = 632
ROWS_LAST = N_NODES - (NS - 1) * ROWS_A

_mesh = plsc.VectorSubcoreMesh(core_axis_name="c", subcore_axis_name="s")


@functools.partial(
    pl.kernel,
    out_type=jax.ShapeDtypeStruct((NC, N_NODES, D_FEAT), jnp.float32),
    mesh=_mesh,
    scratch_types=[
        pltpu.VMEM((NR, GS), jnp.int32),
        pltpu.VMEM((NGS, GS), jnp.int32),
        pltpu.VMEM((NGS, GS), jnp.float32),
        pltpu.VMEM((2, 128, D_FEAT), jnp.float32),
        pltpu.VMEM_SHARED((N_NODES, D_FEAT), jnp.float32),
        pltpu.SemaphoreType.DMA,
        pltpu.SemaphoreType.DMA,
    ],
)
def _sc_aggregate(x_hbm, src_hbm, dst_hbm, w_hbm, out_hbm,
                  src_v, dst_v, w_v, rows_v, acc_sh, gsem, ssem):
    c = lax.axis_index("c")
    s = lax.axis_index("s")
    wid = c * NS + s
    rbase = s * ROWS_A

    zero16 = jnp.zeros((16,), jnp.float32)

    def _zero_row(i, carry):
        for j in range(NVREG):
            rows_v[0, i, pl.ds(j * 16, 16)] = zero16
        return carry

    lax.fori_loop(0, 128, _zero_row, 0)
    for t in range(4):
        pltpu.sync_copy(rows_v.at[0, pl.ds(0, 128)],
                        acc_sh.at[pl.ds(rbase + t * 128, 128)])

    @pl.when(s < NS - 1)
    def _zero_rest():
        pltpu.sync_copy(rows_v.at[0, pl.ds(0, ROWS_A - 512)],
                        acc_sh.at[pl.ds(rbase + 512, ROWS_A - 512)])

    @pl.when(s == NS - 1)
    def _zero_rest_last():
        pltpu.sync_copy(rows_v.at[0, pl.ds(0, ROWS_LAST - 512)],
                        acc_sh.at[pl.ds(rbase + 512, ROWS_LAST - 512)])

    plsc.subcore_barrier()

    bcast_idx = [jnp.full((16, 1), k, jnp.int32) for k in range(16)]
    gdims = lax.GatherDimensionNumbers(
        offset_dims=(), collapsed_slice_dims=(0,), start_index_map=(0,))

    def _bcast(wv, k):
        return lax.gather(wv, bcast_idx[k], dimension_numbers=gdims,
                          slice_sizes=(1,),
                          mode=lax.GatherScatterMode.PROMISE_IN_BOUNDS)

    def _scale16(p, ebase, wv, ks):
        for k in ks:
            wk = _bcast(wv, k)
            e = ebase + k
            for j in range(NVREG):
                rows_v[p, e, pl.ds(j * 16, 16)] = (
                    rows_v[p, e, pl.ds(j * 16, 16)] * wk)

    def _scale_stream(p, l):
        def _scale_group(g, acc2):
            wv = w_v[l, pl.ds(g * 16, 16)]
            _scale16(p, g * 16, wv, range(16))
            return acc2

        lax.fori_loop(0, NFULL, _scale_group, 0)
        wv_t = w_v[l, pl.ds(GS - 16, 16)]
        _scale16(p, GS - 16, wv_t, range(16 - NTAIL, 16))

    pltpu.sync_copy(src_hbm.at[wid], src_v)
    pltpu.async_copy(x_hbm.at[src_v.at[0]], rows_v.at[0, pl.ds(0, GS)],
                     gsem)
    pltpu.async_copy(x_hbm.at[src_v.at[1]], rows_v.at[1, pl.ds(0, GS)],
                     gsem)

    NPAIR = NR // 2

    def _pair(h, inner):
        ci = h // (NGS // 2)
        l0 = 2 * (h % (NGS // 2))
        l1 = l0 + 1
        k0 = 2 * h
        k1 = k0 + 1

        @pl.when(l0 == 0)
        def _load_idx_chunk():
            pltpu.sync_copy(dst_hbm.at[wid, pl.ds(ci * NGS, NGS)], dst_v)
            pltpu.sync_copy(w_hbm.at[wid, pl.ds(ci * NGS, NGS)], w_v)

        pltpu.make_async_copy(x_hbm.at[src_v.at[k0]],
                              rows_v.at[0, pl.ds(0, GS)], gsem).wait()
        _scale_stream(0, l0)
        pltpu.sync_copy(rows_v.at[0, pl.ds(0, GS)],
                        acc_sh.at[dst_v.at[l0]], add=True)

        @pl.when(h < NPAIR - 1)
        def _prefetch0():
            pltpu.async_copy(x_hbm.at[src_v.at[k0 + 2]],
                             rows_v.at[0, pl.ds(0, GS)], gsem)

        pltpu.make_async_copy(x_hbm.at[src_v.at[k1]],
                              rows_v.at[1, pl.ds(0, GS)], gsem).wait()
        _scale_stream(1, l1)
        pltpu.sync_copy(rows_v.at[1, pl.ds(0, GS)],
                        acc_sh.at[dst_v.at[l1]], add=True)

        @pl.when(h < NPAIR - 1)
        def _prefetch1():
            pltpu.async_copy(x_hbm.at[src_v.at[k1 + 2]],
                             rows_v.at[1, pl.ds(0, GS)], gsem)
        return inner

    lax.fori_loop(0, NPAIR, _pair, 0)
    plsc.subcore_barrier()

    for t in range(4):
        pltpu.sync_copy(acc_sh.at[pl.ds(rbase + t * 128, 128)],
                        out_hbm.at[c, pl.ds(rbase + t * 128, 128)])

    @pl.when(s < NS - 1)
    def _drain_rest():
        pltpu.sync_copy(acc_sh.at[pl.ds(rbase + 512, ROWS_A - 512)],
                        out_hbm.at[c, pl.ds(rbase + 512, ROWS_A - 512)])

    @pl.when(s == NS - 1)
    def _drain_rest_last():
        pltpu.sync_copy(acc_sh.at[pl.ds(rbase + 512, ROWS_LAST - 512)],
                        out_hbm.at[c, pl.ds(rbase + 512, ROWS_LAST - 512)])


_TC_BLOCK = 2000


def _tc_body(p0_ref, p1_ref, w_ref, b_ref, o_ref):
    acc = p0_ref[...] + p1_ref[...]
    y = jnp.dot(acc, w_ref[...], preferred_element_type=jnp.float32)
    o_ref[...] = jnp.maximum(y + b_ref[...], 0.0)


_tc_finish = pl.pallas_call(
    _tc_body,
    grid=(N_NODES // _TC_BLOCK,),
    in_specs=[
        pl.BlockSpec((_TC_BLOCK, D_FEAT), lambda i: (i, 0)),
        pl.BlockSpec((_TC_BLOCK, D_FEAT), lambda i: (i, 0)),
        pl.BlockSpec((D_FEAT, D_FEAT), lambda i: (0, 0)),
        pl.BlockSpec((1, D_FEAT), lambda i: (0, 0)),
    ],
    out_specs=pl.BlockSpec((_TC_BLOCK, D_FEAT), lambda i: (i, 0)),
    out_shape=jax.ShapeDtypeStruct((N_NODES, D_FEAT), jnp.float32),
)


def kernel(node_features, edge_index, edge_weight, W, b):
    ei = edge_index.astype(jnp.int32)
    dst3d = ei[0].reshape(NW, NR, GS)
    src3d = ei[1].reshape(NW, NR, GS)
    w3d = edge_weight.reshape(NW, NR, GS)
    partials = _sc_aggregate(node_features, src3d, dst3d, w3d)
    return _tc_finish(partials[0], partials[1], W, b.reshape(1, D_FEAT))
